# baseline (speedup 1.0000x reference)
import functools

import jax
import jax.numpy as jnp
from jax.experimental import pallas as pl
from jax.experimental.pallas import tpu as pltpu


def _decoder_body(sst_ref, sen_ref, sblk8_ref, snch_ref, nseg_ref,
                  qc_ref, fx_ref, lxy_ref,
                  kw0_ref, kb0_ref, kw1_ref, kb1_ref,
                  pw0_ref, pb0_ref, pw1_ref, pb1_ref, out_ref, *,
                  bn, r2, kh):
    b = pl.program_id(0)

    qc = qc_ref[:]
    xp = qc @ kw0_ref[0:2, :]
    kw1 = kw1_ref[:]
    kb1 = kb1_ref[:]
    kb0 = kb0_ref[:]
    w0y = kw0_ref[2:4, :]
    eye2 = (jax.lax.broadcasted_iota(jnp.int32, (2, 2), 0)
            == jax.lax.broadcasted_iota(jnp.int32, (2, 2), 1)
            ).astype(jnp.float32)
    cm = jnp.concatenate([eye2, w0y.T], axis=0)
    qx = qc[:, 0:1]
    qy = qc[:, 1:2]
    bq = qc.shape[0]
    cin = kw1.shape[1]
    lane = jax.lax.broadcasted_iota(jnp.int32, (1, bn), 1)

    def seg_step(s, carry):
        st = sst_ref[b, s]
        en = sen_ref[b, s]
        st_blk8 = sblk8_ref[b, s]
        nch = snch_ref[b, s]

        def chunk_step(i, carry):
            acc, cnt = carry
            off = (st_blk8 + i * (bn // 8)) * 8
            fc = fx_ref[pl.ds(off, bn), :]
            ltxy = lxy_ref[pl.ds(off, bn), 0:2]
            cmb = jax.lax.dot_general(
                cm, ltxy, (((1,), (1,)), ((), ())),
                precision=jax.lax.Precision.HIGHEST,
                preferred_element_type=jnp.float32)
            gidx = lane + off
            inb = (gidx >= st) & (gidx < en)
            dx = qx - cmb[0:1, :]
            dy = qy - cmb[1:2, :]
            dist2 = dx * dx + dy * dy
            mask = ((dist2 <= r2) & inb).astype(jnp.float32)
            cnt = cnt + jnp.sum(mask, axis=1, keepdims=True)
            ypT = cmb[2:, :] + kb0.T
            for j in range(kh):
                g = jax.nn.gelu(xp[:, j:j + 1] + ypT[j:j + 1, :]) * mask
                acc = acc + jnp.dot(g, fc * kw1[j:j + 1, :],
                                    preferred_element_type=jnp.float32)
            acc = acc + jnp.dot(mask, fc * kb1,
                                preferred_element_type=jnp.float32)
            return acc, cnt

        return jax.lax.fori_loop(0, nch, chunk_step, carry)

    acc0 = jnp.zeros((bq, cin), jnp.float32)
    cnt0 = jnp.zeros((bq, 1), jnp.float32)
    acc, cnt = jax.lax.fori_loop(0, nseg_ref[b], seg_step, (acc0, cnt0))
    dec = acc / jnp.maximum(cnt, 1.0)
    h = jax.nn.gelu(jnp.dot(dec, pw0_ref[:], preferred_element_type=jnp.float32)
                    + pb0_ref[:])
    out_ref[:] = jnp.dot(h, pw1_ref[:],
                         preferred_element_type=jnp.float32) + pb1_ref[:]


def _decode_one(ltc, f, qc, K_W0, K_b0, K_W1, K_b1, P_W0, P_b0, P_W1, P_b1,
                radius):
    N = ltc.shape[0]
    Q = qc.shape[0]
    kh = K_W0.shape[1]
    cin = K_W1.shape[1]
    cout = P_W1.shape[1]

    BQ = 400 if Q % 400 == 0 else min(Q, 8)
    BN = 128
    GS = 30
    pad_c = max(1, int(radius * GS) + 1)

    qrow = jnp.clip((qc[:, 0] * GS).astype(jnp.int32), 0, GS - 1)
    qcol = jnp.clip((qc[:, 1] * GS).astype(jnp.int32), 0, GS - 1)
    lrow = jnp.clip((ltc[:, 0] * GS).astype(jnp.int32), 0, GS - 1)
    lcol = jnp.clip((ltc[:, 1] * GS).astype(jnp.int32), 0, GS - 1)
    snake_col = jnp.where(qrow % 2 == 0, qcol, GS - 1 - qcol)
    qkey_s, qperm = jax.lax.sort_key_val(
        qrow * GS + snake_col, jnp.arange(Q, dtype=jnp.int32))
    lcell = lrow * GS + lcol
    lperm = jnp.argsort(lcell)
    qcs = qc[qperm]
    qrow_s = qkey_s // GS
    scol_s = qkey_s % GS
    qcol_s = jnp.where(qrow_s % 2 == 0, scol_s, GS - 1 - scol_s)
    lts = ltc[lperm]
    fs = f[lperm]
    loff = jnp.searchsorted(lcell[lperm],
                            jnp.arange(GS * GS + 1, dtype=jnp.int32),
                            side='left').astype(jnp.int32)

    qpad = (-Q) % BQ
    if qpad:
        qcs = jnp.concatenate(
            [qcs, jnp.full((qpad, qcs.shape[1]), 1e6, qcs.dtype)], axis=0)
        qrow_s = jnp.concatenate(
            [qrow_s, jnp.full((qpad,), GS - 1, qrow_s.dtype)], axis=0)
        qcol_s = jnp.concatenate(
            [qcol_s, jnp.full((qpad,), GS - 1, qcol_s.dtype)], axis=0)
    npad = ((-N) % 8) + BN
    lts = jnp.concatenate(
        [lts, jnp.full((npad, lts.shape[1]), 1e6, lts.dtype)], axis=0)
    fs = jnp.concatenate(
        [fs, jnp.zeros((npad, fs.shape[1]), fs.dtype)], axis=0)
    lxy = jnp.concatenate(
        [lts, jnp.zeros((lts.shape[0], 8 - lts.shape[1]), jnp.float32)],
        axis=1)
    Qp = qcs.shape[0]
    nb = Qp // BQ

    qrow_blk = qrow_s.reshape(nb, BQ)
    qcol_blk = qcol_s.reshape(nb, BQ)
    rlo = jnp.clip(jnp.min(qrow_blk, axis=1) - pad_c, 0, GS - 1)
    rhi = jnp.clip(jnp.max(qrow_blk, axis=1) + pad_c, 0, GS - 1)
    clo = jnp.clip(jnp.min(qcol_blk, axis=1) - pad_c, 0, GS - 1)
    chi = jnp.clip(jnp.max(qcol_blk, axis=1) + pad_c, 0, GS - 1)
    nseg = (rhi - rlo + 1).astype(jnp.int32)
    rows = rlo[:, None] + jnp.arange(GS, dtype=jnp.int32)[None, :]
    valid = rows <= rhi[:, None]
    rows_c = jnp.minimum(rows, GS - 1)
    cell_lo = rows_c * GS + clo[:, None]
    cell_hi = rows_c * GS + chi[:, None] + 1
    sst = jnp.where(valid, loff[cell_lo], 0).astype(jnp.int32)
    sen = jnp.where(valid, loff[cell_hi], 0).astype(jnp.int32)
    sblk8 = (sst // 8).astype(jnp.int32)
    snch = jnp.where(sen > sst, (sen - sblk8 * 8 + BN - 1) // BN,
                     0).astype(jnp.int32)

    body = functools.partial(_decoder_body, bn=BN, r2=radius * radius, kh=kh)
    grid_spec = pltpu.PrefetchScalarGridSpec(
        num_scalar_prefetch=5,
        grid=(nb,),
        in_specs=[
            pl.BlockSpec((BQ, 2), lambda i, *_: (i, 0)),
            pl.BlockSpec((fs.shape[0], cin), lambda i, *_: (0, 0)),
            pl.BlockSpec((lxy.shape[0], 8), lambda i, *_: (0, 0)),
            pl.BlockSpec(K_W0.shape, lambda i, *_: (0, 0)),
            pl.BlockSpec((1, kh), lambda i, *_: (0, 0)),
            pl.BlockSpec(K_W1.shape, lambda i, *_: (0, 0)),
            pl.BlockSpec((1, cin), lambda i, *_: (0, 0)),
            pl.BlockSpec(P_W0.shape, lambda i, *_: (0, 0)),
            pl.BlockSpec((1, P_W0.shape[1]), lambda i, *_: (0, 0)),
            pl.BlockSpec(P_W1.shape, lambda i, *_: (0, 0)),
            pl.BlockSpec((1, cout), lambda i, *_: (0, 0)),
        ],
        out_specs=pl.BlockSpec((BQ, cout), lambda i, *_: (i, 0)),
    )
    out_sorted = pl.pallas_call(
        body,
        grid_spec=grid_spec,
        out_shape=jax.ShapeDtypeStruct((Qp, cout), jnp.float32),
    )(sst, sen, sblk8, snch, nseg,
      qcs, fs, lxy, K_W0, K_b0[None, :], K_W1, K_b1[None, :],
      P_W0, P_b0[None, :], P_W1, P_b1[None, :])
    invperm = jnp.zeros((Q,), jnp.int32).at[qperm].set(
        jnp.arange(Q, dtype=jnp.int32))
    return out_sorted[invperm]


def kernel(latent_tokens_coord, rndata, query_coord, K_W0, K_b0, K_W1, K_b1,
           P_W0, P_b0, P_W1, P_b1):
    B = query_coord.shape[0]
    radius = 0.033
    outs = []
    for b in range(B):
        outs.append(_decode_one(
            latent_tokens_coord, rndata[b], query_coord[b],
            K_W0, K_b0, K_W1, K_b1, P_W0, P_b0, P_W1, P_b1, radius))
    return jnp.stack(outs, axis=0)


# speedup vs baseline: 1.0545x; 1.0545x over previous
import functools

import jax
import jax.numpy as jnp
from jax.experimental import pallas as pl
from jax.experimental.pallas import tpu as pltpu


def _decoder_body(sst_ref, sen_ref, sblk8_ref, snch_ref, nseg_ref,
                  qc_ref, fx_ref, lxy_ref,
                  kw0_ref, kb0_ref, kw1_ref, kb1_ref,
                  pw0_ref, pb0_ref, pw1_ref, pb1_ref, out_ref, *,
                  bn, r2, kh):
    b = pl.program_id(0)

    qc = qc_ref[:]
    xp = qc @ kw0_ref[0:2, :]
    kw1 = kw1_ref[:]
    kb1 = kb1_ref[:]
    kb0 = kb0_ref[:]
    w0y = kw0_ref[2:4, :]
    eye2 = (jax.lax.broadcasted_iota(jnp.int32, (2, 2), 0)
            == jax.lax.broadcasted_iota(jnp.int32, (2, 2), 1)
            ).astype(jnp.float32)
    cm = jnp.concatenate([eye2, w0y.T], axis=0)
    qx = qc[:, 0:1]
    qy = qc[:, 1:2]
    bq = qc.shape[0]
    cin = kw1.shape[1]
    lane = jax.lax.broadcasted_iota(jnp.int32, (1, bn), 1)

    def seg_step(s, carry):
        st = sst_ref[b, s]
        en = sen_ref[b, s]
        st_blk8 = sblk8_ref[b, s]
        nch = snch_ref[b, s]

        def chunk_step(i, carry):
            acc, cnt = carry
            off = (st_blk8 + i * (bn // 8)) * 8
            fc = fx_ref[pl.ds(off, bn), :]
            ltxy = lxy_ref[pl.ds(off, bn), 0:2]
            cmb = jax.lax.dot_general(
                cm, ltxy, (((1,), (1,)), ((), ())),
                precision=jax.lax.Precision.HIGHEST,
                preferred_element_type=jnp.float32)
            gidx = lane + off
            inb = (gidx >= st) & (gidx < en)
            dx = qx - cmb[0:1, :]
            dy = qy - cmb[1:2, :]
            dist2 = dx * dx + dy * dy
            mask = ((dist2 <= r2) & inb).astype(jnp.float32)
            cnt = cnt + jnp.sum(mask, axis=1, keepdims=True)
            ypT = cmb[2:, :] + kb0.T
            for j in range(kh):
                g = jax.nn.gelu(xp[:, j:j + 1] + ypT[j:j + 1, :]) * mask
                acc = acc + jnp.dot(g, fc * kw1[j:j + 1, :],
                                    preferred_element_type=jnp.float32)
            acc = acc + jnp.dot(mask, fc * kb1,
                                preferred_element_type=jnp.float32)
            return acc, cnt

        return jax.lax.fori_loop(0, nch, chunk_step, carry)

    acc0 = jnp.zeros((bq, cin), jnp.float32)
    cnt0 = jnp.zeros((bq, 1), jnp.float32)
    acc, cnt = jax.lax.fori_loop(0, nseg_ref[b], seg_step, (acc0, cnt0))
    dec = acc / jnp.maximum(cnt, 1.0)
    h = jax.nn.gelu(jnp.dot(dec, pw0_ref[:], preferred_element_type=jnp.float32)
                    + pb0_ref[:])
    out_ref[:] = jnp.dot(h, pw1_ref[:],
                         preferred_element_type=jnp.float32) + pb1_ref[:]


def _decode_one(ltc, f, qc, K_W0, K_b0, K_W1, K_b1, P_W0, P_b0, P_W1, P_b1,
                radius):
    N = ltc.shape[0]
    Q = qc.shape[0]
    kh = K_W0.shape[1]
    cin = K_W1.shape[1]
    cout = P_W1.shape[1]

    BQ = 400 if Q % 400 == 0 else min(Q, 8)
    BN = 128
    GS = 30
    pad_c = max(1, int(radius * GS) + 1)

    qrow = jnp.clip((qc[:, 0] * GS).astype(jnp.int32), 0, GS - 1)
    qcol = jnp.clip((qc[:, 1] * GS).astype(jnp.int32), 0, GS - 1)
    lrow = jnp.clip((ltc[:, 0] * GS).astype(jnp.int32), 0, GS - 1)
    lcol = jnp.clip((ltc[:, 1] * GS).astype(jnp.int32), 0, GS - 1)
    snake_col = jnp.where(qrow % 2 == 0, qcol, GS - 1 - qcol)
    qkey_s, qperm = jax.lax.sort_key_val(
        qrow * GS + snake_col, jnp.arange(Q, dtype=jnp.int32))
    lcell = lrow * GS + lcol
    lperm = jnp.argsort(lcell)
    qcs = qc[qperm]
    qrow_s = qkey_s // GS
    scol_s = qkey_s % GS
    qcol_s = jnp.where(qrow_s % 2 == 0, scol_s, GS - 1 - scol_s)
    lts = ltc[lperm]
    fs = f[lperm]
    loff = jnp.searchsorted(lcell[lperm],
                            jnp.arange(GS * GS + 1, dtype=jnp.int32),
                            side='left').astype(jnp.int32)

    qpad = (-Q) % BQ
    if qpad:
        qcs = jnp.concatenate(
            [qcs, jnp.full((qpad, qcs.shape[1]), 1e6, qcs.dtype)], axis=0)
        qrow_s = jnp.concatenate(
            [qrow_s, jnp.full((qpad,), GS - 1, qrow_s.dtype)], axis=0)
        qcol_s = jnp.concatenate(
            [qcol_s, jnp.full((qpad,), GS - 1, qcol_s.dtype)], axis=0)
    npad = ((-N) % 8) + BN
    lts = jnp.concatenate(
        [lts, jnp.full((npad, lts.shape[1]), 1e6, lts.dtype)], axis=0)
    fs = jnp.concatenate(
        [fs, jnp.zeros((npad, fs.shape[1]), fs.dtype)], axis=0)
    lxy = jnp.concatenate(
        [lts, jnp.zeros((lts.shape[0], 8 - lts.shape[1]), jnp.float32)],
        axis=1)
    Qp = qcs.shape[0]
    nb = Qp // BQ

    qrow_blk = qrow_s.reshape(nb, BQ)
    qcol_blk = qcol_s.reshape(nb, BQ)
    rlo = jnp.clip(jnp.min(qrow_blk, axis=1) - pad_c, 0, GS - 1)
    rhi = jnp.clip(jnp.max(qrow_blk, axis=1) + pad_c, 0, GS - 1)
    clo = jnp.clip(jnp.min(qcol_blk, axis=1) - pad_c, 0, GS - 1)
    chi = jnp.clip(jnp.max(qcol_blk, axis=1) + pad_c, 0, GS - 1)
    nseg = (rhi - rlo + 1).astype(jnp.int32)
    rows = rlo[:, None] + jnp.arange(GS, dtype=jnp.int32)[None, :]
    valid = rows <= rhi[:, None]
    rows_c = jnp.minimum(rows, GS - 1)
    cell_lo = rows_c * GS + clo[:, None]
    cell_hi = rows_c * GS + chi[:, None] + 1
    sst = jnp.where(valid, loff[cell_lo], 0).astype(jnp.int32)
    sen = jnp.where(valid, loff[cell_hi], 0).astype(jnp.int32)
    sblk8 = (sst // 8).astype(jnp.int32)
    snch = jnp.where(sen > sst, (sen - sblk8 * 8 + BN - 1) // BN,
                     0).astype(jnp.int32)

    body = functools.partial(_decoder_body, bn=BN, r2=radius * radius, kh=kh)
    grid_spec = pltpu.PrefetchScalarGridSpec(
        num_scalar_prefetch=5,
        grid=(nb,),
        in_specs=[
            pl.BlockSpec((BQ, 2), lambda i, *_: (i, 0)),
            pl.BlockSpec((fs.shape[0], cin), lambda i, *_: (0, 0)),
            pl.BlockSpec((lxy.shape[0], 8), lambda i, *_: (0, 0)),
            pl.BlockSpec(K_W0.shape, lambda i, *_: (0, 0)),
            pl.BlockSpec((1, kh), lambda i, *_: (0, 0)),
            pl.BlockSpec(K_W1.shape, lambda i, *_: (0, 0)),
            pl.BlockSpec((1, cin), lambda i, *_: (0, 0)),
            pl.BlockSpec(P_W0.shape, lambda i, *_: (0, 0)),
            pl.BlockSpec((1, P_W0.shape[1]), lambda i, *_: (0, 0)),
            pl.BlockSpec(P_W1.shape, lambda i, *_: (0, 0)),
            pl.BlockSpec((1, cout), lambda i, *_: (0, 0)),
        ],
        out_specs=pl.BlockSpec((BQ, cout), lambda i, *_: (i, 0)),
    )
    out_sorted = pl.pallas_call(
        body,
        grid_spec=grid_spec,
        out_shape=jax.ShapeDtypeStruct((Qp, cout), jnp.float32),
    )(sst, sen, sblk8, snch, nseg,
      qcs, fs, lxy, K_W0, K_b0[None, :], K_W1, K_b1[None, :],
      P_W0, P_b0[None, :], P_W1, P_b1[None, :])
    return jnp.zeros((Q, cout), jnp.float32).at[qperm].set(out_sorted[:Q])


def kernel(latent_tokens_coord, rndata, query_coord, K_W0, K_b0, K_W1, K_b1,
           P_W0, P_b0, P_W1, P_b1):
    B = query_coord.shape[0]
    radius = 0.033
    outs = []
    for b in range(B):
        outs.append(_decode_one(
            latent_tokens_coord, rndata[b], query_coord[b],
            K_W0, K_b0, K_W1, K_b1, P_W0, P_b0, P_W1, P_b1, radius))
    return jnp.stack(outs, axis=0)
